# Initial kernel scaffold; baseline (speedup 1.0000x reference)
import jax
import jax.numpy as jnp
from jax.experimental import pallas as pl


def kernel(x, adj, W1, b1, W2, b2):
    raise NotImplementedError("write your pallas kernel here")


# speedup vs baseline: 4566.7800x; 4566.7800x over previous
import jax
import jax.numpy as jnp
from jax.experimental import pallas as pl


def _gcn_body(x_ref, adj_ref, W1_ref, b1_ref, W2_ref, b2_ref, out_ref):
    adj = adj_ref[...]
    deg = jnp.sum(adj, axis=0)
    dinv = jnp.where(deg > 0.0, jax.lax.rsqrt(jnp.where(deg > 0.0, deg, 1.0)), 0.0)
    dcol = dinv[:, None]

    xw = jnp.dot(x_ref[...], W1_ref[...], preferred_element_type=jnp.float32)
    t1 = jax.lax.dot_general(
        adj, xw * dcol, (((0,), (0,)), ((), ())), preferred_element_type=jnp.float32
    )
    h = jnp.maximum(t1 * dcol + b1_ref[...], 0.0)

    hw = jnp.dot(h, W2_ref[...], preferred_element_type=jnp.float32)
    t2 = jax.lax.dot_general(
        adj, hw * dcol, (((0,), (0,)), ((), ())), preferred_element_type=jnp.float32
    )
    out_ref[...] = t2 * dcol + b2_ref[...]


def kernel(x, adj, W1, b1, W2, b2):
    n = x.shape[0]
    return pl.pallas_call(
        _gcn_body,
        out_shape=jax.ShapeDtypeStruct((n, W2.shape[1]), x.dtype),
    )(x, adj, W1, b1.reshape(1, -1), W2, b2.reshape(1, -1))
